# Initial kernel scaffold; baseline (speedup 1.0000x reference)
import jax
import jax.numpy as jnp
from jax.experimental import pallas as pl


def kernel(x, points):
    raise NotImplementedError("write your pallas kernel here")


# speedup vs baseline: 418.6203x; 418.6203x over previous
import dataclasses
import functools

import jax
import jax.numpy as jnp
from jax.experimental import pallas as pl
from jax.experimental.pallas import tpu as pltpu
from jax.experimental.pallas import tpu_sc as plsc

_N_CHANNELS = 96
_N_REGIONS = 128
_BOUND = 2.5
_SCALE = _N_REGIONS / (2.0 * _BOUND)
_SHIFT = _BOUND * _SCALE

_ROWS = None

_LANES = 16
_CHUNK = 6272


def _sc_pwlu(x2, a_flat, d_flat, coff):
    rows, cols = x2.shape
    n_chunks = cols // _CHUNK
    mesh = plsc.VectorSubcoreMesh(core_axis_name="c", subcore_axis_name="s")
    cp = pltpu.CompilerParams()
    if "needs_layout_passes" in pltpu.CompilerParams.__dataclass_fields__:
        cp = dataclasses.replace(cp, needs_layout_passes=False)

    @functools.partial(
        pl.kernel,
        compiler_params=cp,
        out_type=jax.ShapeDtypeStruct((rows, cols), jnp.float32),
        mesh=mesh,
        scratch_types=[
            pltpu.VMEM((_N_CHANNELS * _N_REGIONS,), jnp.float32),
            pltpu.VMEM((_N_CHANNELS * _N_REGIONS,), jnp.float32),
            pltpu.SemaphoreType.DMA,
        ],
    )
    def run(x_hbm, a_hbm, d_hbm, coff_hbm, o_hbm, a_v, d_v, sem):
        pltpu.async_copy(a_hbm, a_v, sem).wait()
        pltpu.async_copy(d_hbm, d_v, sem).wait()

        def body(x_v, c_v, o_v):
            cv = c_v[0, :]

            @pl.loop(0, _CHUNK, step=_LANES)
            def _(i):
                xv = x_v[0, pl.ds(i, _LANES)]
                xn = xv * _SCALE + _SHIFT
                cl = jnp.clip(xn, 0.0, float(_N_REGIONS - 1))
                f = cl.astype(jnp.int32) + cv
                av = plsc.load_gather(a_v, [f])
                dv = plsc.load_gather(d_v, [f])
                o_v[0, pl.ds(i, _LANES)] = av + xn * dv

        pltpu.emit_pipeline(
            body,
            grid=(rows, n_chunks),
            in_specs=[
                pl.BlockSpec((1, _CHUNK), lambda i, j: (i, j)),
                pl.BlockSpec((1, _LANES), lambda i, j: (i, 0)),
            ],
            out_specs=[pl.BlockSpec((1, _CHUNK), lambda i, j: (i, j))],
            core_axis_name=("c", "s"),
            dimension_semantics=(pltpu.PARALLEL, pltpu.PARALLEL),
        )(x_hbm, coff_hbm, o_hbm)

    return run(x2, a_flat, d_flat, coff)


def kernel(x, points):
    b, c, h, w = x.shape
    rows = b * c
    cols = h * w
    d = points[:, 1:] - points[:, :-1]
    r = jnp.arange(_N_REGIONS, dtype=jnp.float32)
    a = points[:, :-1] - r[None, :] * d
    a_flat = a.reshape(-1)
    d_flat = d.reshape(-1)
    coff = (jnp.arange(rows, dtype=jnp.int32) % c) * _N_REGIONS
    coff = jnp.broadcast_to(coff[:, None], (rows, _LANES))
    x2 = x.reshape(rows, cols)
    out = _sc_pwlu(x2, a_flat, d_flat, coff)
    return out.reshape(x.shape)


# speedup vs baseline: 1413.6177x; 3.3768x over previous
import dataclasses
import functools

import jax
import jax.numpy as jnp
from jax.experimental import pallas as pl
from jax.experimental.pallas import tpu as pltpu
from jax.experimental.pallas import tpu_sc as plsc

_N_CHANNELS = 96
_N_REGIONS = 128
_BOUND = 2.5
_SCALE = _N_REGIONS / (2.0 * _BOUND)
_SHIFT = _BOUND * _SCALE

_ROWS = None

_LANES = 16
_CHUNK = 25088


def _sc_pwlu(x2, a_flat, d_flat, coff):
    rows, cols = x2.shape
    n_chunks = cols // _CHUNK
    mesh = plsc.VectorSubcoreMesh(core_axis_name="c", subcore_axis_name="s")
    cp = pltpu.CompilerParams()
    if "needs_layout_passes" in pltpu.CompilerParams.__dataclass_fields__:
        cp = dataclasses.replace(cp, needs_layout_passes=False)

    @functools.partial(
        pl.kernel,
        compiler_params=cp,
        out_type=jax.ShapeDtypeStruct((rows, cols), jnp.float32),
        mesh=mesh,
        scratch_types=[
            pltpu.VMEM((_N_CHANNELS * _N_REGIONS,), jnp.float32),
            pltpu.VMEM((_N_CHANNELS * _N_REGIONS,), jnp.float32),
            pltpu.SemaphoreType.DMA,
        ],
    )
    def run(x_hbm, a_hbm, d_hbm, coff_hbm, o_hbm, a_v, d_v, sem):
        pltpu.async_copy(a_hbm, a_v, sem).wait()
        pltpu.async_copy(d_hbm, d_v, sem).wait()

        def body(x_v, c_v, o_v):
            cv = c_v[0, :]

            @plsc.parallel_loop(0, _CHUNK, _LANES, unroll=8)
            def _(i):
                xv = x_v[0, pl.ds(i, _LANES)]
                xn = xv * _SCALE + _SHIFT
                cl = jnp.clip(xn, 0.0, float(_N_REGIONS - 1))
                f = cl.astype(jnp.int32) + cv
                av = plsc.load_gather(a_v, [f])
                dv = plsc.load_gather(d_v, [f])
                o_v[0, pl.ds(i, _LANES)] = av + xn * dv

        pltpu.emit_pipeline(
            body,
            grid=(rows, n_chunks),
            in_specs=[
                pl.BlockSpec((1, _CHUNK), lambda i, j: (i, j)),
                pl.BlockSpec((1, _LANES), lambda i, j: (i, 0)),
            ],
            out_specs=[pl.BlockSpec((1, _CHUNK), lambda i, j: (i, j))],
            core_axis_name=("c", "s"),
            dimension_semantics=(pltpu.PARALLEL, pltpu.PARALLEL),
        )(x_hbm, coff_hbm, o_hbm)

    return run(x2, a_flat, d_flat, coff)


def kernel(x, points):
    b, c, h, w = x.shape
    rows = b * c
    cols = h * w
    d = points[:, 1:] - points[:, :-1]
    r = jnp.arange(_N_REGIONS, dtype=jnp.float32)
    a = points[:, :-1] - r[None, :] * d
    a_flat = a.reshape(-1)
    d_flat = d.reshape(-1)
    coff = (jnp.arange(rows, dtype=jnp.int32) % c) * _N_REGIONS
    coff = jnp.broadcast_to(coff[:, None], (rows, _LANES))
    x2 = x.reshape(rows, cols)
    out = _sc_pwlu(x2, a_flat, d_flat, coff)
    return out.reshape(x.shape)


# speedup vs baseline: 1726.2760x; 1.2212x over previous
import dataclasses
import functools

import jax
import jax.numpy as jnp
from jax.experimental import pallas as pl
from jax.experimental.pallas import tpu as pltpu
from jax.experimental.pallas import tpu_sc as plsc

_N_CHANNELS = 96
_N_REGIONS = 128
_BOUND = 2.5
_SCALE = _N_REGIONS / (2.0 * _BOUND)
_SHIFT = _BOUND * _SCALE

_LANES = 16
_W = 224
_SLAB = 224
_CHUNK_ROWS = 56
_NBUF = 2

_NC = 2
_NS = 16


def _sc_pwlu(x2, a_flat, d_flat):
    rows, cols = x2.shape
    n_workers = _NC * _NS
    rows_per_worker = rows // n_workers
    slabs_per_worker = rows_per_worker // _SLAB
    chunks_per_slab = _SLAB // _CHUNK_ROWS

    mesh = plsc.VectorSubcoreMesh(core_axis_name="c", subcore_axis_name="s")
    cp = pltpu.CompilerParams()
    if "needs_layout_passes" in pltpu.CompilerParams.__dataclass_fields__:
        cp = dataclasses.replace(cp, needs_layout_passes=False)
    if "use_tc_tiling_on_sc" in pltpu.CompilerParams.__dataclass_fields__:
        cp = dataclasses.replace(cp, use_tc_tiling_on_sc=True)

    @functools.partial(
        pl.kernel,
        out_type=jax.ShapeDtypeStruct((rows, cols), jnp.float32),
        mesh=mesh,
        compiler_params=cp,
        scratch_types=[
            pltpu.VMEM((_N_CHANNELS * _N_REGIONS,), jnp.float32),
            pltpu.VMEM((_N_CHANNELS * _N_REGIONS,), jnp.float32),
            pltpu.VMEM((_NBUF, _CHUNK_ROWS, _W), jnp.float32),
            pltpu.VMEM((_NBUF, _CHUNK_ROWS, _W), jnp.float32),
            pltpu.SemaphoreType.DMA,
            pltpu.SemaphoreType.DMA,
            pltpu.SemaphoreType.DMA,
        ],
    )
    def run(x_hbm, a_hbm, d_hbm, o_hbm, a_v, d_v, x_b, o_b, sem_t, sem_i, sem_o):
        pltpu.async_copy(a_hbm, a_v, sem_t).wait()
        pltpu.async_copy(d_hbm, d_v, sem_t).wait()

        cid = jax.lax.axis_index("c")
        sid = jax.lax.axis_index("s")
        wid = sid * _NC + cid
        row0 = wid * rows_per_worker
        slab0 = wid * slabs_per_worker
        c0 = jax.lax.rem(slab0, _N_CHANNELS)

        n_chunks = slabs_per_worker * chunks_per_slab

        def chunk_row(k):
            return row0 + k * _CHUNK_ROWS

        def start_in(k, buf):
            return pltpu.make_async_copy(
                x_hbm.at[pl.ds(chunk_row(k), _CHUNK_ROWS), :],
                x_b.at[buf],
                sem_i,
            )

        def start_out(k, buf):
            return pltpu.make_async_copy(
                o_b.at[buf],
                o_hbm.at[pl.ds(chunk_row(k), _CHUNK_ROWS), :],
                sem_o,
            )

        def compute(buf, coff):
            @plsc.parallel_loop(0, _CHUNK_ROWS, 1)
            def _(r):
                @plsc.parallel_loop(0, _W, _LANES, unroll=7)
                def _(j):
                    xv = x_b[buf, r, pl.ds(j, _LANES)]
                    xn = xv * _SCALE + _SHIFT
                    cl = jnp.clip(xn, 0.0, float(_N_REGIONS - 1))
                    f = cl.astype(jnp.int32) + coff
                    av = plsc.load_gather(a_v, [f])
                    dv = plsc.load_gather(d_v, [f])
                    o_b[buf, r, pl.ds(j, _LANES)] = av + xn * dv

        start_in(0, 0).start()
        start_in(1, 1).start()

        def coff_vec(c):
            return jnp.broadcast_to((c * _N_REGIONS).astype(jnp.int32), (_LANES,))

        @pl.loop(0, n_chunks, step=_NBUF, init_carry=c0)
        def _(k, c):
            for b in range(_NBUF):
                kk = k + b
                start_in(kk, b).wait()
                @pl.when(kk >= _NBUF)
                def _():
                    start_out(kk - _NBUF, b).wait()

                compute(b, coff_vec(c))
                start_out(kk, b).start()

                @pl.when(kk + _NBUF < n_chunks)
                def _():
                    start_in(kk + _NBUF, b).start()

                bump = jax.lax.rem(kk, chunks_per_slab) == (chunks_per_slab - 1)
                c = jnp.where(bump, c + 1, c)
                c = jnp.where(c >= _N_CHANNELS, c - _N_CHANNELS, c)
            return c

        for t in range(_NBUF):
            b = (n_chunks - _NBUF + t) % _NBUF
            start_out(n_chunks - _NBUF + t, b).wait()

    return run(x2, a_flat, d_flat)


def kernel(x, points):
    b, c, h, w = x.shape
    d = points[:, 1:] - points[:, :-1]
    r = jnp.arange(_N_REGIONS, dtype=jnp.float32)
    a = points[:, :-1] - r[None, :] * d
    a_flat = a.reshape(-1)
    d_flat = d.reshape(-1)
    x2 = x.reshape(b * c * h, w)
    out = _sc_pwlu(x2, a_flat, d_flat)
    return out.reshape(x.shape)


# speedup vs baseline: 2696.5372x; 1.5621x over previous
import dataclasses
import functools

import jax
import jax.numpy as jnp
from jax.experimental import pallas as pl
from jax.experimental.pallas import tpu as pltpu
from jax.experimental.pallas import tpu_sc as plsc

_N_CHANNELS = 96
_N_REGIONS = 128
_BOUND = 2.5
_SCALE = _N_REGIONS / (2.0 * _BOUND)
_SHIFT = _BOUND * _SCALE

_LANES = 16
_W = 224
_SLAB = 224
_CHUNK_ROWS = 56
_NBUF = 2

_NC = 2
_NS = 16


def _sc_pwlu(x2, a_flat, d_flat):
    rows, cols = x2.shape
    n_workers = _NC * _NS
    rows_per_worker = rows // n_workers
    slabs_per_worker = rows_per_worker // _SLAB
    chunks_per_slab = _SLAB // _CHUNK_ROWS

    mesh = plsc.VectorSubcoreMesh(core_axis_name="c", subcore_axis_name="s")
    cp = pltpu.CompilerParams()
    if "needs_layout_passes" in pltpu.CompilerParams.__dataclass_fields__:
        cp = dataclasses.replace(cp, needs_layout_passes=False)
    if "use_tc_tiling_on_sc" in pltpu.CompilerParams.__dataclass_fields__:
        cp = dataclasses.replace(cp, use_tc_tiling_on_sc=True)

    @functools.partial(
        pl.kernel,
        out_type=jax.ShapeDtypeStruct((rows, cols), jnp.float32),
        mesh=mesh,
        compiler_params=cp,
        scratch_types=[
            pltpu.VMEM((_N_CHANNELS * _N_REGIONS,), jnp.float32),
            pltpu.VMEM((_N_CHANNELS * _N_REGIONS,), jnp.float32),
            pltpu.VMEM((_NBUF, _CHUNK_ROWS, _W), jnp.float32),
            pltpu.VMEM((_NBUF, _CHUNK_ROWS, _W), jnp.float32),
            pltpu.SemaphoreType.DMA,
            pltpu.SemaphoreType.DMA,
            pltpu.SemaphoreType.DMA,
        ],
    )
    def run(x_hbm, a_hbm, d_hbm, o_hbm, a_v, d_v, x_b, o_b, sem_t, sem_i, sem_o):
        pltpu.async_copy(a_hbm, a_v, sem_t).wait()
        pltpu.async_copy(d_hbm, d_v, sem_t).wait()

        cid = jax.lax.axis_index("c")
        sid = jax.lax.axis_index("s")
        wid = sid * _NC + cid
        row0 = wid * rows_per_worker
        slab0 = wid * slabs_per_worker
        c0 = jax.lax.rem(slab0, _N_CHANNELS)

        n_chunks = slabs_per_worker * chunks_per_slab

        def chunk_row(k):
            return row0 + k * _CHUNK_ROWS

        def start_in(k, buf):
            return pltpu.make_async_copy(
                x_hbm.at[pl.ds(chunk_row(k), _CHUNK_ROWS), :],
                x_b.at[buf],
                sem_i,
            )

        def start_out(k, buf):
            return pltpu.make_async_copy(
                o_b.at[buf],
                o_hbm.at[pl.ds(chunk_row(k), _CHUNK_ROWS), :],
                sem_o,
            )

        def compute(buf, coff):
            @plsc.parallel_loop(0, _CHUNK_ROWS, 1)
            def _(r):
                @plsc.parallel_loop(0, _W, _LANES, unroll=_W // _LANES)
                def _(j):
                    xv = x_b[buf, r, pl.ds(j, _LANES)]
                    xn = xv * _SCALE + _SHIFT
                    cl = jnp.clip(xn, 0.0, float(_N_REGIONS - 1))
                    f = cl.astype(jnp.int32) + coff
                    av = plsc.load_gather(a_v, [f])
                    dv = plsc.load_gather(d_v, [f])
                    o_b[buf, r, pl.ds(j, _LANES)] = av + xn * dv

        start_in(0, 0).start()
        start_in(1, 1).start()

        def coff_vec(c):
            return jnp.broadcast_to((c * _N_REGIONS).astype(jnp.int32), (_LANES,))

        @pl.loop(0, n_chunks, step=_NBUF, init_carry=c0)
        def _(k, c):
            for b in range(_NBUF):
                kk = k + b
                start_in(kk, b).wait()
                @pl.when(kk >= _NBUF)
                def _():
                    start_out(kk - _NBUF, b).wait()

                compute(b, coff_vec(c))
                start_out(kk, b).start()

                @pl.when(kk + _NBUF < n_chunks)
                def _():
                    start_in(kk + _NBUF, b).start()

                bump = jax.lax.rem(kk, chunks_per_slab) == (chunks_per_slab - 1)
                c = jnp.where(bump, c + 1, c)
                c = jnp.where(c >= _N_CHANNELS, c - _N_CHANNELS, c)
            return c

        for t in range(_NBUF):
            b = (n_chunks - _NBUF + t) % _NBUF
            start_out(n_chunks - _NBUF + t, b).wait()

    return run(x2, a_flat, d_flat)


def kernel(x, points):
    b, c, h, w = x.shape
    d = points[:, 1:] - points[:, :-1]
    r = jnp.arange(_N_REGIONS, dtype=jnp.float32)
    a = points[:, :-1] - r[None, :] * d
    a_flat = a.reshape(-1)
    d_flat = d.reshape(-1)
    x2 = x.reshape(b * c * h, w)
    out = _sc_pwlu(x2, a_flat, d_flat)
    return out.reshape(x.shape)
